# baseline (speedup 1.0000x reference)
import functools

import jax
import jax.numpy as jnp
from jax import lax
from jax.experimental import pallas as pl
from jax.experimental.pallas import tpu as pltpu
from jax.experimental.pallas import tpu_sc as plsc

B, L, N, H = 32, 8192, 64, 128
NUM_REL = 8
NC, NS = 2, 16
TB = 4
M = TB * N


def _sc_gather_body(emb_hbm, ids_hbm, out_hbm, idx_v, rows_v, sem):
    wid = lax.axis_index("s") * NC + lax.axis_index("c")
    pltpu.sync_copy(ids_hbm.at[wid], idx_v)
    for i in range(N // 16):
        sl = pl.ds(i * 16, 16)
        idx_v[sl] = idx_v[sl] + wid * L
    pltpu.async_copy(emb_hbm.at[idx_v], rows_v, sem).wait()
    pltpu.sync_copy(rows_v, out_hbm.at[wid])


def _sc_gather(emb_flat, num_ids):
    mesh = plsc.VectorSubcoreMesh(core_axis_name="c", subcore_axis_name="s")
    k = functools.partial(
        pl.kernel,
        mesh=mesh,
        out_type=jax.ShapeDtypeStruct((B, N, H), jnp.float32),
        scratch_types=[
            pltpu.VMEM((N,), jnp.int32),
            pltpu.VMEM((N, H), jnp.float32),
            pltpu.SemaphoreType.DMA,
        ],
    )(_sc_gather_body)
    return k(emb_flat, num_ids)


def _nt(m, w):
    return lax.dot_general(m, w, (((1,), (1,)), ((), ())),
                           preferred_element_type=jnp.float32)


GS = 2
CH = B // (TB * GS)


def _dense_body(x_ref, ncol_ref, nrow_ref, rcol_ref, wa_ref, ba_ref,
                wf_ref, bf_ref, wr_ref, out_ref):
    bf16 = jnp.bfloat16
    ii = lax.broadcasted_iota(jnp.int32, (M, M), 0)
    jj = lax.broadcasted_iota(jnp.int32, (M, M), 1)
    valid = jnp.where((ii != jj) & ((ii >> 6) == (jj >> 6)), 1.0, 0.0)
    wa = wa_ref[...]
    bf = bf_ref[...]
    ba = ba_ref[...]

    for c in range(CH):
        x = x_ref[0, c]
        nrow = nrow_ref[0, c]
        ncol = ncol_ref[0, c]
        rcol = rcol_ref[0, c]

        lin = jnp.sum(x * wa, axis=1, keepdims=True) + ba
        a = (1.0 / (N - 1)) / (1.0 + jnp.exp(-lin))

        ax = x * a
        x1 = jnp.where(rcol > 0.5, ax, 0.0)
        x0 = ax - x1
        x0b = x0.astype(bf16)
        x1b = x1.astype(bf16)

        G = jnp.where(ncol > nrow, valid, 0.0).astype(bf16)

        A0 = jnp.dot(G, x0b, preferred_element_type=jnp.float32)
        A1 = jnp.dot(G, x1b, preferred_element_type=jnp.float32)
        cs0 = jnp.sum(x0.reshape(TB, N, H), axis=1, keepdims=True)
        cs1 = jnp.sum(x1.reshape(TB, N, H), axis=1, keepdims=True)
        S0 = jnp.broadcast_to(cs0, (TB, N, H)).reshape(M, H)
        S1 = jnp.broadcast_to(cs1, (TB, N, H)).reshape(M, H)
        B0 = (S0 - x0 - A0).astype(bf16)
        B1 = (S1 - x1 - A1).astype(bf16)
        A0b = A0.astype(bf16)
        A1b = A1.astype(bf16)
        msg0 = _nt(A0b, wr_ref[4]) + _nt(A1b, wr_ref[5]) + _nt(B0, wr_ref[0]) + _nt(B1, wr_ref[1])
        msg1 = _nt(A0b, wr_ref[6]) + _nt(A1b, wr_ref[7]) + _nt(B0, wr_ref[2]) + _nt(B1, wr_ref[3])
        msg = jnp.where(rcol > 0.5, msg1, msg0)

        y = _nt(x, wf_ref[...]) + bf + msg
        out_ref[0, c] = jnp.maximum(y, 0.0)


def _dense(x, ncol, nrow, rcol, wa, ba, wf, bf, wr, interpret=False):
    out = pl.pallas_call(
        _dense_body,
        grid=(GS,),
        in_specs=[
            pl.BlockSpec((1, CH, M, H), lambda i: (i, 0, 0, 0)),
            pl.BlockSpec((1, CH, M, 1), lambda i: (i, 0, 0, 0)),
            pl.BlockSpec((1, CH, 1, M), lambda i: (i, 0, 0, 0)),
            pl.BlockSpec((1, CH, M, 1), lambda i: (i, 0, 0, 0)),
            pl.BlockSpec((1, H), lambda i: (0, 0)),
            pl.BlockSpec((1, 1), lambda i: (0, 0)),
            pl.BlockSpec((H, H), lambda i: (0, 0)),
            pl.BlockSpec((1, H), lambda i: (0, 0)),
            pl.BlockSpec((NUM_REL, H, H), lambda i: (0, 0, 0)),
        ],
        out_specs=pl.BlockSpec((1, CH, M, H), lambda i: (i, 0, 0, 0)),
        out_shape=jax.ShapeDtypeStruct((GS, CH, M, H), jnp.float32),
        interpret=interpret,
    )(x.reshape(GS, CH, M, H), ncol, nrow, rcol, wa, ba, wf, bf, wr)
    return out.reshape(B, N, H)


def _prep(numbers, is_response, W_alpha, b_alpha, W_f, b_f, W_r):
    isr = is_response.astype(jnp.float32)
    return (numbers.reshape(GS, CH, M, 1), numbers.reshape(GS, CH, 1, M),
            isr.reshape(GS, CH, M, 1), W_alpha.reshape(1, H),
            b_alpha.reshape(1, 1), W_f, b_f.reshape(1, H),
            W_r.astype(jnp.bfloat16))


def kernel(word_emb, num_ids, is_response, numbers, W_alpha, b_alpha, W_f, b_f, W_r):
    init_emb = _sc_gather(word_emb.reshape(B * L, H), num_ids)
    args = _prep(numbers, is_response, W_alpha, b_alpha, W_f, b_f, W_r)
    return _dense(init_emb, *args)


# speedup vs baseline: 1.0304x; 1.0304x over previous
import functools

import jax
import jax.numpy as jnp
from jax import lax
from jax.experimental import pallas as pl
from jax.experimental.pallas import tpu as pltpu
from jax.experimental.pallas import tpu_sc as plsc

B, L, N, H = 32, 8192, 64, 128
NUM_REL = 8
NC, NS = 2, 16
TB = 4
M = TB * N


def _sc_gather_body(emb_hbm, ids_hbm, out_hbm, idx_v, rows_v, sem):
    wid = lax.axis_index("s") * NC + lax.axis_index("c")
    pltpu.sync_copy(ids_hbm.at[wid], idx_v)
    for i in range(N // 16):
        sl = pl.ds(i * 16, 16)
        idx_v[sl] = idx_v[sl] + wid * L
    pltpu.async_copy(emb_hbm.at[idx_v], rows_v, sem).wait()
    pltpu.sync_copy(rows_v, out_hbm.at[wid])


def _sc_gather(emb_flat, num_ids):
    mesh = plsc.VectorSubcoreMesh(core_axis_name="c", subcore_axis_name="s")
    k = functools.partial(
        pl.kernel,
        mesh=mesh,
        out_type=jax.ShapeDtypeStruct((B, N, H), jnp.float32),
        scratch_types=[
            pltpu.VMEM((N,), jnp.int32),
            pltpu.VMEM((N, H), jnp.float32),
            pltpu.SemaphoreType.DMA,
        ],
    )(_sc_gather_body)
    return k(emb_flat, num_ids)


def _nt(m, w):
    return lax.dot_general(m, w, (((1,), (1,)), ((), ())),
                           preferred_element_type=jnp.float32)


GS = 2
CH = B // (TB * GS)


def _dense_body(x_ref, ncol_ref, nrow_ref, rcol_ref, wa_ref, ba_ref,
                wf_ref, bf_ref, wr_ref, out_ref):
    bf16 = jnp.bfloat16
    ii = lax.broadcasted_iota(jnp.int32, (M, M), 0)
    jj = lax.broadcasted_iota(jnp.int32, (M, M), 1)
    valid = jnp.where((ii != jj) & ((ii >> 6) == (jj >> 6)), 1.0, 0.0)
    wa = wa_ref[...]
    bf = bf_ref[...]
    ba = ba_ref[...]

    for c in range(CH):
        x = x_ref[0, c]
        nrow = nrow_ref[0, c]
        ncol = ncol_ref[0, c]
        rcol = rcol_ref[0, c]

        lin = jnp.sum(x * wa, axis=1, keepdims=True) + ba
        a = (1.0 / (N - 1)) / (1.0 + jnp.exp(-lin))

        ax = x * a
        x1 = jnp.where(rcol > 0.5, ax, 0.0)
        x0 = ax - x1

        G = jnp.where(ncol > nrow, valid, 0.0)

        A0 = jnp.dot(G, x0, preferred_element_type=jnp.float32)
        A1 = jnp.dot(G, x1, preferred_element_type=jnp.float32)
        cs0 = jnp.sum(x0.reshape(TB, N, H), axis=1, keepdims=True)
        cs1 = jnp.sum(x1.reshape(TB, N, H), axis=1, keepdims=True)
        S0 = jnp.broadcast_to(cs0, (TB, N, H)).reshape(M, H)
        S1 = jnp.broadcast_to(cs1, (TB, N, H)).reshape(M, H)
        B0 = S0 - x0 - A0
        B1 = S1 - x1 - A1
        msg0 = _nt(A0, wr_ref[4]) + _nt(A1, wr_ref[5]) + _nt(B0, wr_ref[0]) + _nt(B1, wr_ref[1])
        msg1 = _nt(A0, wr_ref[6]) + _nt(A1, wr_ref[7]) + _nt(B0, wr_ref[2]) + _nt(B1, wr_ref[3])
        msg = jnp.where(rcol > 0.5, msg1, msg0)

        y = _nt(x, wf_ref[...]) + bf + msg
        out_ref[0, c] = jnp.maximum(y, 0.0)


def _dense(x, ncol, nrow, rcol, wa, ba, wf, bf, wr, interpret=False):
    out = pl.pallas_call(
        _dense_body,
        grid=(GS,),
        in_specs=[
            pl.BlockSpec((1, CH, M, H), lambda i: (i, 0, 0, 0)),
            pl.BlockSpec((1, CH, M, 1), lambda i: (i, 0, 0, 0)),
            pl.BlockSpec((1, CH, 1, M), lambda i: (i, 0, 0, 0)),
            pl.BlockSpec((1, CH, M, 1), lambda i: (i, 0, 0, 0)),
            pl.BlockSpec((1, H), lambda i: (0, 0)),
            pl.BlockSpec((1, 1), lambda i: (0, 0)),
            pl.BlockSpec((H, H), lambda i: (0, 0)),
            pl.BlockSpec((1, H), lambda i: (0, 0)),
            pl.BlockSpec((NUM_REL, H, H), lambda i: (0, 0, 0)),
        ],
        out_specs=pl.BlockSpec((1, CH, M, H), lambda i: (i, 0, 0, 0)),
        out_shape=jax.ShapeDtypeStruct((GS, CH, M, H), jnp.float32),
        interpret=interpret,
    )(x.reshape(GS, CH, M, H), ncol, nrow, rcol, wa, ba, wf, bf, wr)
    return out.reshape(B, N, H)


def _prep(numbers, is_response, W_alpha, b_alpha, W_f, b_f, W_r):
    isr = is_response.astype(jnp.float32)
    return (numbers.reshape(GS, CH, M, 1), numbers.reshape(GS, CH, 1, M),
            isr.reshape(GS, CH, M, 1), W_alpha.reshape(1, H),
            b_alpha.reshape(1, 1), W_f, b_f.reshape(1, H), W_r)


def kernel(word_emb, num_ids, is_response, numbers, W_alpha, b_alpha, W_f, b_f, W_r):
    init_emb = _sc_gather(word_emb.reshape(B * L, H), num_ids)
    args = _prep(numbers, is_response, W_alpha, b_alpha, W_f, b_f, W_r)
    return _dense(init_emb, *args)
